# baseline (speedup 1.0000x reference)
import functools

import jax
import jax.numpy as jnp
from jax import lax
from jax.experimental import pallas as pl
from jax.experimental.pallas import tpu as pltpu
from jax.experimental.pallas import tpu_sc as plsc

BATCH = 16384
EMBED_DIM = 128
ID_DIM = 64
NUM_FEATS = 20
HIDDEN = 128
NUM_PAIRS = 500000

_NC = 2
_NS = 16
_NW = _NC * _NS
_B_PER_W = BATCH // _NW
_IDX_CHUNK = 128


def _gather_pairs_sc(pair_ids, table2):
    mesh = plsc.VectorSubcoreMesh(core_axis_name="c", subcore_axis_name="s")

    @functools.partial(
        pl.kernel,
        mesh=mesh,
        out_type=jax.ShapeDtypeStruct((BATCH, 2 * ID_DIM), jnp.float32),
        scratch_types=[
            pltpu.VMEM((_B_PER_W,), jnp.int32),
            pltpu.VMEM((_B_PER_W, 2 * ID_DIM), jnp.float32),
            pltpu.SemaphoreType.DMA,
        ],
    )
    def gather_kernel(idx_hbm, table_hbm, out_hbm, idx_v, rows_v, sem):
        wid = lax.axis_index("s") * _NC + lax.axis_index("c")
        base = wid * _B_PER_W
        pltpu.sync_copy(idx_hbm.at[pl.ds(base, _B_PER_W)], idx_v)
        copies = []
        for j in range(_B_PER_W // _IDX_CHUNK):
            sl = pl.ds(j * _IDX_CHUNK, _IDX_CHUNK)
            copies.append(
                pltpu.async_copy(table_hbm.at[idx_v.at[sl]], rows_v.at[sl], sem)
            )
        for c in copies:
            c.wait()
        pltpu.sync_copy(rows_v, out_hbm.at[pl.ds(base, _B_PER_W)])

    return gather_kernel(pair_ids, table2)


def _mlp_body(pair_ref, half_ref, feat_ref, w1a_ref, w1b_ref, b1_ref, w2_ref,
              b2_ref, out_ref):
    pair = pair_ref[...]
    h = half_ref[...]
    emb = pair[:, :ID_DIM] * (1.0 - h) + pair[:, ID_DIM:] * h
    z = jnp.dot(emb, w1a_ref[...], preferred_element_type=jnp.float32)
    z += jnp.dot(feat_ref[...], w1b_ref[...], preferred_element_type=jnp.float32)
    z = jnp.maximum(z + b1_ref[...], 0.0)
    out = jnp.dot(z, w2_ref[...], preferred_element_type=jnp.float32) + b2_ref[...]
    norm = jnp.sqrt(jnp.sum(out * out, axis=1, keepdims=True))
    out_ref[...] = out / jnp.maximum(norm, 1e-12)


_BB = 2048


def _mlp_tc(pairs, half, feats, w1a, w1b, b1, w2, b2):
    grid = (BATCH // _BB,)
    return pl.pallas_call(
        _mlp_body,
        grid=grid,
        in_specs=[
            pl.BlockSpec((_BB, 2 * ID_DIM), lambda i: (i, 0)),
            pl.BlockSpec((_BB, 1), lambda i: (i, 0)),
            pl.BlockSpec((_BB, NUM_FEATS), lambda i: (i, 0)),
            pl.BlockSpec((ID_DIM, HIDDEN), lambda i: (0, 0)),
            pl.BlockSpec((NUM_FEATS, HIDDEN), lambda i: (0, 0)),
            pl.BlockSpec((1, HIDDEN), lambda i: (0, 0)),
            pl.BlockSpec((HIDDEN, EMBED_DIM), lambda i: (0, 0)),
            pl.BlockSpec((1, EMBED_DIM), lambda i: (0, 0)),
        ],
        out_specs=pl.BlockSpec((_BB, EMBED_DIM), lambda i: (i, 0)),
        out_shape=jax.ShapeDtypeStruct((BATCH, EMBED_DIM), jnp.float32),
    )(pairs, half, feats, w1a, w1b, b1, w2, b2)


def kernel(customer_ids, numerical_features, emb_table, W1, b1, W2, b2):
    ids = customer_ids.astype(jnp.int32)
    table2 = emb_table.reshape(NUM_PAIRS, 2 * ID_DIM)
    pair_ids = ids // 2
    half = (ids % 2).astype(jnp.float32).reshape(BATCH, 1)
    pairs = _gather_pairs_sc(pair_ids, table2)
    w1a = W1[:ID_DIM]
    w1b = W1[ID_DIM:]
    return _mlp_tc(
        pairs,
        half,
        numerical_features,
        w1a,
        w1b,
        b1.reshape(1, HIDDEN),
        W2,
        b2.reshape(1, EMBED_DIM),
    )


# speedup vs baseline: 1.7474x; 1.7474x over previous
import functools

import jax
import jax.numpy as jnp
from jax import lax
from jax.experimental import pallas as pl
from jax.experimental.pallas import tpu as pltpu
from jax.experimental.pallas import tpu_sc as plsc

BATCH = 16384
EMBED_DIM = 128
ID_DIM = 64
NUM_FEATS = 20
HIDDEN = 128
NUM_CUST = 1000000

_NC = 2
_NS = 16
_NW = _NC * _NS
_B_PER_W = BATCH // _NW
_IDX_CHUNK = 128

_RL = 4096
_RGRID = pl.cdiv(NUM_CUST, _RL)
_NPROWS = _RGRID * (_RL // 2)


def _relayout_body(t2_ref, out_ref):
    x = t2_ref[...]
    lo = x[:, : _RL // 2].T
    hi = x[:, _RL // 2 :].T
    out_ref[...] = jnp.concatenate([lo, hi], axis=1)


def _relayout_tc(table_t):
    return pl.pallas_call(
        _relayout_body,
        grid=(_RGRID,),
        in_specs=[pl.BlockSpec((ID_DIM, _RL), lambda i: (0, i))],
        out_specs=pl.BlockSpec((_RL // 2, 2 * ID_DIM), lambda i: (i, 0)),
        out_shape=jax.ShapeDtypeStruct((_NPROWS, 2 * ID_DIM), jnp.float32),
    )(table_t)


def _gather_pairs_sc(pair_ids, table2):
    mesh = plsc.VectorSubcoreMesh(core_axis_name="c", subcore_axis_name="s")

    @functools.partial(
        pl.kernel,
        mesh=mesh,
        out_type=jax.ShapeDtypeStruct((BATCH, 2 * ID_DIM), jnp.float32),
        scratch_types=[
            pltpu.VMEM((_B_PER_W,), jnp.int32),
            pltpu.VMEM((_B_PER_W, 2 * ID_DIM), jnp.float32),
            pltpu.SemaphoreType.DMA,
        ],
    )
    def gather_kernel(idx_hbm, table_hbm, out_hbm, idx_v, rows_v, sem):
        wid = lax.axis_index("s") * _NC + lax.axis_index("c")
        base = wid * _B_PER_W
        pltpu.sync_copy(idx_hbm.at[pl.ds(base, _B_PER_W)], idx_v)
        copies = []
        for j in range(_B_PER_W // _IDX_CHUNK):
            sl = pl.ds(j * _IDX_CHUNK, _IDX_CHUNK)
            copies.append(
                pltpu.async_copy(table_hbm.at[idx_v.at[sl]], rows_v.at[sl], sem)
            )
        for c in copies:
            c.wait()
        pltpu.sync_copy(rows_v, out_hbm.at[pl.ds(base, _B_PER_W)])

    return gather_kernel(pair_ids, table2)


def _mlp_body(pair_ref, half_ref, feat_ref, w1a_ref, w1b_ref, b1_ref, w2_ref,
              b2_ref, out_ref):
    pair = pair_ref[...]
    h = half_ref[...]
    emb = pair[:, :ID_DIM] * (1.0 - h) + pair[:, ID_DIM:] * h
    z = jnp.dot(emb, w1a_ref[...], preferred_element_type=jnp.float32)
    z += jnp.dot(feat_ref[...], w1b_ref[...], preferred_element_type=jnp.float32)
    z = jnp.maximum(z + b1_ref[...], 0.0)
    out = jnp.dot(z, w2_ref[...], preferred_element_type=jnp.float32) + b2_ref[...]
    norm = jnp.sqrt(jnp.sum(out * out, axis=1, keepdims=True))
    out_ref[...] = out / jnp.maximum(norm, 1e-12)


_BB = 2048


def _mlp_tc(pairs, half, feats, w1a, w1b, b1, w2, b2):
    grid = (BATCH // _BB,)
    return pl.pallas_call(
        _mlp_body,
        grid=grid,
        in_specs=[
            pl.BlockSpec((_BB, 2 * ID_DIM), lambda i: (i, 0)),
            pl.BlockSpec((_BB, 1), lambda i: (i, 0)),
            pl.BlockSpec((_BB, NUM_FEATS), lambda i: (i, 0)),
            pl.BlockSpec((ID_DIM, HIDDEN), lambda i: (0, 0)),
            pl.BlockSpec((NUM_FEATS, HIDDEN), lambda i: (0, 0)),
            pl.BlockSpec((1, HIDDEN), lambda i: (0, 0)),
            pl.BlockSpec((HIDDEN, EMBED_DIM), lambda i: (0, 0)),
            pl.BlockSpec((1, EMBED_DIM), lambda i: (0, 0)),
        ],
        out_specs=pl.BlockSpec((_BB, EMBED_DIM), lambda i: (i, 0)),
        out_shape=jax.ShapeDtypeStruct((BATCH, EMBED_DIM), jnp.float32),
    )(pairs, half, feats, w1a, w1b, b1, w2, b2)


def kernel(customer_ids, numerical_features, emb_table, W1, b1, W2, b2):
    ids = customer_ids.astype(jnp.int32)
    table_t = emb_table.T
    table2 = _relayout_tc(table_t)
    pair_ids = (ids // _RL) * (_RL // 2) + ids % (_RL // 2)
    half = ((ids % _RL) >= _RL // 2).astype(jnp.float32).reshape(BATCH, 1)
    pairs = _gather_pairs_sc(pair_ids, table2)
    w1a = W1[:ID_DIM]
    w1b = W1[ID_DIM:]
    return _mlp_tc(
        pairs,
        half,
        numerical_features,
        w1a,
        w1b,
        b1.reshape(1, HIDDEN),
        W2,
        b2.reshape(1, EMBED_DIM),
    )


# speedup vs baseline: 2.1266x; 1.2170x over previous
import functools

import jax
import jax.numpy as jnp
from jax import lax
from jax.experimental import pallas as pl
from jax.experimental.pallas import tpu as pltpu
from jax.experimental.pallas import tpu_sc as plsc

BATCH = 16384
EMBED_DIM = 128
ID_DIM = 64
NUM_FEATS = 20
HIDDEN = 128
NUM_CUST = 1000000

_NC = 2
_NS = 16
_NW = _NC * _NS
_B_PER_W = BATCH // _NW
_IDX_CHUNK = 128

_RL = 8192
_RGRID = pl.cdiv(NUM_CUST, _RL)
_NPROWS = _RGRID * (_RL // 2)


def _relayout_body(t2_ref, out_ref):
    x = t2_ref[...]
    lo = x[:, : _RL // 2].T
    hi = x[:, _RL // 2 :].T
    out_ref[...] = jnp.concatenate([lo, hi], axis=1)


def _relayout_tc(table_t):
    return pl.pallas_call(
        _relayout_body,
        grid=(_RGRID,),
        in_specs=[pl.BlockSpec((ID_DIM, _RL), lambda i: (0, i))],
        out_specs=pl.BlockSpec((_RL // 2, 2 * ID_DIM), lambda i: (i, 0)),
        out_shape=jax.ShapeDtypeStruct((_NPROWS, 2 * ID_DIM), jnp.float32),
    )(table_t)


def _gather_pairs_sc(pair_ids, table2):
    mesh = plsc.VectorSubcoreMesh(core_axis_name="c", subcore_axis_name="s")

    @functools.partial(
        pl.kernel,
        mesh=mesh,
        out_type=jax.ShapeDtypeStruct((BATCH, 2 * ID_DIM), jnp.float32),
        scratch_types=[
            pltpu.VMEM((_B_PER_W,), jnp.int32),
            pltpu.VMEM((_B_PER_W, 2 * ID_DIM), jnp.float32),
            pltpu.SemaphoreType.DMA,
        ],
    )
    def gather_kernel(idx_hbm, table_hbm, out_hbm, idx_v, rows_v, sem):
        wid = lax.axis_index("s") * _NC + lax.axis_index("c")
        base = wid * _B_PER_W
        pltpu.sync_copy(idx_hbm.at[pl.ds(base, _B_PER_W)], idx_v)
        copies = []
        for j in range(_B_PER_W // _IDX_CHUNK):
            sl = pl.ds(j * _IDX_CHUNK, _IDX_CHUNK)
            copies.append(
                pltpu.async_copy(table_hbm.at[idx_v.at[sl]], rows_v.at[sl], sem)
            )
        for c in copies:
            c.wait()
        pltpu.sync_copy(rows_v, out_hbm.at[pl.ds(base, _B_PER_W)])

    return gather_kernel(pair_ids, table2)


def _mlp_body(pair_ref, half_ref, feat_ref, w1a_ref, w1b_ref, b1_ref, w2_ref,
              b2_ref, out_ref):
    pair = pair_ref[...]
    h = half_ref[...]
    emb = pair[:, :ID_DIM] * (1.0 - h) + pair[:, ID_DIM:] * h
    z = jnp.dot(emb, w1a_ref[...], preferred_element_type=jnp.float32)
    z += jnp.dot(feat_ref[...], w1b_ref[...], preferred_element_type=jnp.float32)
    z = jnp.maximum(z + b1_ref[...], 0.0)
    out = jnp.dot(z, w2_ref[...], preferred_element_type=jnp.float32) + b2_ref[...]
    norm = jnp.sqrt(jnp.sum(out * out, axis=1, keepdims=True))
    out_ref[...] = out / jnp.maximum(norm, 1e-12)


_BB = 2048


def _mlp_tc(pairs, half, feats, w1a, w1b, b1, w2, b2):
    grid = (BATCH // _BB,)
    return pl.pallas_call(
        _mlp_body,
        grid=grid,
        in_specs=[
            pl.BlockSpec((_BB, 2 * ID_DIM), lambda i: (i, 0)),
            pl.BlockSpec((_BB, 1), lambda i: (i, 0)),
            pl.BlockSpec((_BB, NUM_FEATS), lambda i: (i, 0)),
            pl.BlockSpec((ID_DIM, HIDDEN), lambda i: (0, 0)),
            pl.BlockSpec((NUM_FEATS, HIDDEN), lambda i: (0, 0)),
            pl.BlockSpec((1, HIDDEN), lambda i: (0, 0)),
            pl.BlockSpec((HIDDEN, EMBED_DIM), lambda i: (0, 0)),
            pl.BlockSpec((1, EMBED_DIM), lambda i: (0, 0)),
        ],
        out_specs=pl.BlockSpec((_BB, EMBED_DIM), lambda i: (i, 0)),
        out_shape=jax.ShapeDtypeStruct((BATCH, EMBED_DIM), jnp.float32),
    )(pairs, half, feats, w1a, w1b, b1, w2, b2)


def kernel(customer_ids, numerical_features, emb_table, W1, b1, W2, b2):
    ids = customer_ids.astype(jnp.int32)
    table_t = emb_table.T
    table2 = _relayout_tc(table_t)
    pair_ids = (ids // _RL) * (_RL // 2) + ids % (_RL // 2)
    half = ((ids % _RL) >= _RL // 2).astype(jnp.float32).reshape(BATCH, 1)
    pairs = _gather_pairs_sc(pair_ids, table2)
    w1a = W1[:ID_DIM]
    w1b = W1[ID_DIM:]
    return _mlp_tc(
        pairs,
        half,
        numerical_features,
        w1a,
        w1b,
        b1.reshape(1, HIDDEN),
        W2,
        b2.reshape(1, EMBED_DIM),
    )


# speedup vs baseline: 2.3940x; 1.1257x over previous
import functools

import jax
import jax.numpy as jnp
from jax import lax
from jax.experimental import pallas as pl
from jax.experimental.pallas import tpu as pltpu
from jax.experimental.pallas import tpu_sc as plsc

BATCH = 16384
EMBED_DIM = 128
ID_DIM = 64
NUM_FEATS = 20
HIDDEN = 128
NUM_CUST = 1000000

_NC = 2
_NS = 16
_NW = _NC * _NS
_B_PER_W = BATCH // _NW
_IDX_CHUNK = 128

_RL = 16384
_RGRID = pl.cdiv(NUM_CUST, _RL)
_NPROWS = _RGRID * (_RL // 2)


def _relayout_body(t2_ref, out_ref):
    x = t2_ref[...]
    lo = x[:, : _RL // 2].T
    hi = x[:, _RL // 2 :].T
    out_ref[...] = jnp.concatenate([lo, hi], axis=1)


def _relayout_tc(table_t):
    return pl.pallas_call(
        _relayout_body,
        grid=(_RGRID,),
        in_specs=[pl.BlockSpec((ID_DIM, _RL), lambda i: (0, i))],
        out_specs=pl.BlockSpec((_RL // 2, 2 * ID_DIM), lambda i: (i, 0)),
        out_shape=jax.ShapeDtypeStruct((_NPROWS, 2 * ID_DIM), jnp.float32),
    )(table_t)


def _gather_pairs_sc(pair_ids, table2):
    mesh = plsc.VectorSubcoreMesh(core_axis_name="c", subcore_axis_name="s")

    @functools.partial(
        pl.kernel,
        mesh=mesh,
        out_type=jax.ShapeDtypeStruct((BATCH, 2 * ID_DIM), jnp.float32),
        scratch_types=[
            pltpu.VMEM((_B_PER_W,), jnp.int32),
            pltpu.VMEM((_B_PER_W, 2 * ID_DIM), jnp.float32),
            pltpu.SemaphoreType.DMA,
        ],
    )
    def gather_kernel(idx_hbm, table_hbm, out_hbm, idx_v, rows_v, sem):
        wid = lax.axis_index("s") * _NC + lax.axis_index("c")
        base = wid * _B_PER_W
        pltpu.sync_copy(idx_hbm.at[pl.ds(base, _B_PER_W)], idx_v)
        copies = []
        for j in range(_B_PER_W // _IDX_CHUNK):
            sl = pl.ds(j * _IDX_CHUNK, _IDX_CHUNK)
            copies.append(
                pltpu.async_copy(table_hbm.at[idx_v.at[sl]], rows_v.at[sl], sem)
            )
        for c in copies:
            c.wait()
        pltpu.sync_copy(rows_v, out_hbm.at[pl.ds(base, _B_PER_W)])

    return gather_kernel(pair_ids, table2)


def _mlp_body(pair_ref, half_ref, feat_ref, w1a_ref, w1b_ref, b1_ref, w2_ref,
              b2_ref, out_ref):
    pair = pair_ref[...]
    h = half_ref[...]
    emb = pair[:, :ID_DIM] * (1.0 - h) + pair[:, ID_DIM:] * h
    z = jnp.dot(emb, w1a_ref[...], preferred_element_type=jnp.float32)
    z += jnp.dot(feat_ref[...], w1b_ref[...], preferred_element_type=jnp.float32)
    z = jnp.maximum(z + b1_ref[...], 0.0)
    out = jnp.dot(z, w2_ref[...], preferred_element_type=jnp.float32) + b2_ref[...]
    norm = jnp.sqrt(jnp.sum(out * out, axis=1, keepdims=True))
    out_ref[...] = out / jnp.maximum(norm, 1e-12)


_BB = 2048


def _mlp_tc(pairs, half, feats, w1a, w1b, b1, w2, b2):
    grid = (BATCH // _BB,)
    return pl.pallas_call(
        _mlp_body,
        grid=grid,
        in_specs=[
            pl.BlockSpec((_BB, 2 * ID_DIM), lambda i: (i, 0)),
            pl.BlockSpec((_BB, 1), lambda i: (i, 0)),
            pl.BlockSpec((_BB, NUM_FEATS), lambda i: (i, 0)),
            pl.BlockSpec((ID_DIM, HIDDEN), lambda i: (0, 0)),
            pl.BlockSpec((NUM_FEATS, HIDDEN), lambda i: (0, 0)),
            pl.BlockSpec((1, HIDDEN), lambda i: (0, 0)),
            pl.BlockSpec((HIDDEN, EMBED_DIM), lambda i: (0, 0)),
            pl.BlockSpec((1, EMBED_DIM), lambda i: (0, 0)),
        ],
        out_specs=pl.BlockSpec((_BB, EMBED_DIM), lambda i: (i, 0)),
        out_shape=jax.ShapeDtypeStruct((BATCH, EMBED_DIM), jnp.float32),
    )(pairs, half, feats, w1a, w1b, b1, w2, b2)


def kernel(customer_ids, numerical_features, emb_table, W1, b1, W2, b2):
    ids = customer_ids.astype(jnp.int32)
    table_t = emb_table.T
    table2 = _relayout_tc(table_t)
    pair_ids = (ids // _RL) * (_RL // 2) + ids % (_RL // 2)
    half = ((ids % _RL) >= _RL // 2).astype(jnp.float32).reshape(BATCH, 1)
    pairs = _gather_pairs_sc(pair_ids, table2)
    w1a = W1[:ID_DIM]
    w1b = W1[ID_DIM:]
    return _mlp_tc(
        pairs,
        half,
        numerical_features,
        w1a,
        w1b,
        b1.reshape(1, HIDDEN),
        W2,
        b2.reshape(1, EMBED_DIM),
    )


# speedup vs baseline: 2.5157x; 1.0508x over previous
import functools

import jax
import jax.numpy as jnp
from jax import lax
from jax.experimental import pallas as pl
from jax.experimental.pallas import tpu as pltpu
from jax.experimental.pallas import tpu_sc as plsc

BATCH = 16384
EMBED_DIM = 128
ID_DIM = 64
NUM_FEATS = 20
HIDDEN = 128
NUM_CUST = 1000000

_NC = 2
_NS = 16
_NW = _NC * _NS
_B_PER_W = BATCH // _NW
_IDX_CHUNK = 128

_RL = 32768
_RGRID = pl.cdiv(NUM_CUST, _RL)
_NPROWS = _RGRID * (_RL // 2)


def _relayout_body(t2_ref, out_ref):
    x = t2_ref[...]
    lo = x[:, : _RL // 2].T
    hi = x[:, _RL // 2 :].T
    out_ref[...] = jnp.concatenate([lo, hi], axis=1)


def _relayout_tc(table_t):
    return pl.pallas_call(
        _relayout_body,
        grid=(_RGRID,),
        in_specs=[pl.BlockSpec((ID_DIM, _RL), lambda i: (0, i))],
        out_specs=pl.BlockSpec((_RL // 2, 2 * ID_DIM), lambda i: (i, 0)),
        out_shape=jax.ShapeDtypeStruct((_NPROWS, 2 * ID_DIM), jnp.float32),
    )(table_t)


def _gather_pairs_sc(pair_ids, table2):
    mesh = plsc.VectorSubcoreMesh(core_axis_name="c", subcore_axis_name="s")

    @functools.partial(
        pl.kernel,
        mesh=mesh,
        out_type=jax.ShapeDtypeStruct((BATCH, 2 * ID_DIM), jnp.float32),
        scratch_types=[
            pltpu.VMEM((_B_PER_W,), jnp.int32),
            pltpu.VMEM((_B_PER_W, 2 * ID_DIM), jnp.float32),
            pltpu.SemaphoreType.DMA,
        ],
    )
    def gather_kernel(idx_hbm, table_hbm, out_hbm, idx_v, rows_v, sem):
        wid = lax.axis_index("s") * _NC + lax.axis_index("c")
        base = wid * _B_PER_W
        pltpu.sync_copy(idx_hbm.at[pl.ds(base, _B_PER_W)], idx_v)
        copies = []
        for j in range(_B_PER_W // _IDX_CHUNK):
            sl = pl.ds(j * _IDX_CHUNK, _IDX_CHUNK)
            copies.append(
                pltpu.async_copy(table_hbm.at[idx_v.at[sl]], rows_v.at[sl], sem)
            )
        for c in copies:
            c.wait()
        pltpu.sync_copy(rows_v, out_hbm.at[pl.ds(base, _B_PER_W)])

    return gather_kernel(pair_ids, table2)


def _mlp_body(pair_ref, half_ref, feat_ref, w1a_ref, w1b_ref, b1_ref, w2_ref,
              b2_ref, out_ref):
    pair = pair_ref[...]
    h = half_ref[...]
    emb = pair[:, :ID_DIM] * (1.0 - h) + pair[:, ID_DIM:] * h
    z = jnp.dot(emb, w1a_ref[...], preferred_element_type=jnp.float32)
    z += jnp.dot(feat_ref[...], w1b_ref[...], preferred_element_type=jnp.float32)
    z = jnp.maximum(z + b1_ref[...], 0.0)
    out = jnp.dot(z, w2_ref[...], preferred_element_type=jnp.float32) + b2_ref[...]
    norm = jnp.sqrt(jnp.sum(out * out, axis=1, keepdims=True))
    out_ref[...] = out / jnp.maximum(norm, 1e-12)


_BB = 2048


def _mlp_tc(pairs, half, feats, w1a, w1b, b1, w2, b2):
    grid = (BATCH // _BB,)
    return pl.pallas_call(
        _mlp_body,
        grid=grid,
        in_specs=[
            pl.BlockSpec((_BB, 2 * ID_DIM), lambda i: (i, 0)),
            pl.BlockSpec((_BB, 1), lambda i: (i, 0)),
            pl.BlockSpec((_BB, NUM_FEATS), lambda i: (i, 0)),
            pl.BlockSpec((ID_DIM, HIDDEN), lambda i: (0, 0)),
            pl.BlockSpec((NUM_FEATS, HIDDEN), lambda i: (0, 0)),
            pl.BlockSpec((1, HIDDEN), lambda i: (0, 0)),
            pl.BlockSpec((HIDDEN, EMBED_DIM), lambda i: (0, 0)),
            pl.BlockSpec((1, EMBED_DIM), lambda i: (0, 0)),
        ],
        out_specs=pl.BlockSpec((_BB, EMBED_DIM), lambda i: (i, 0)),
        out_shape=jax.ShapeDtypeStruct((BATCH, EMBED_DIM), jnp.float32),
    )(pairs, half, feats, w1a, w1b, b1, w2, b2)


def kernel(customer_ids, numerical_features, emb_table, W1, b1, W2, b2):
    ids = customer_ids.astype(jnp.int32)
    table_t = emb_table.T
    table2 = _relayout_tc(table_t)
    pair_ids = (ids // _RL) * (_RL // 2) + ids % (_RL // 2)
    half = ((ids % _RL) >= _RL // 2).astype(jnp.float32).reshape(BATCH, 1)
    pairs = _gather_pairs_sc(pair_ids, table2)
    w1a = W1[:ID_DIM]
    w1b = W1[ID_DIM:]
    return _mlp_tc(
        pairs,
        half,
        numerical_features,
        w1a,
        w1b,
        b1.reshape(1, HIDDEN),
        W2,
        b2.reshape(1, EMBED_DIM),
    )


# speedup vs baseline: 3.3990x; 1.3511x over previous
import functools

import jax
import jax.numpy as jnp
from jax import lax
from jax.experimental import pallas as pl
from jax.experimental.pallas import tpu as pltpu
from jax.experimental.pallas import tpu_sc as plsc

BATCH = 16384
EMBED_DIM = 128
ID_DIM = 64
NUM_FEATS = 20
HIDDEN = 128
NUM_CUST = 1000000

_NC = 2
_NS = 16
_NW = _NC * _NS
_B_PER_W = BATCH // _NW
_IDX_CHUNK = 128

_RL = 32768
_RGRID = pl.cdiv(NUM_CUST, _RL)
_NPROWS = _RGRID * (_RL // 4)


def _relayout_body(t2_ref, out_ref):
    x = t2_ref[...].astype(jnp.bfloat16)
    bits = lax.bitcast_convert_type(x, jnp.uint16).astype(jnp.uint32)
    w = lax.bitcast_convert_type(
        bits[:, : _RL // 2] | (bits[:, _RL // 2 :] << 16), jnp.int32
    )
    lo = w[:, : _RL // 4].T
    hi = w[:, _RL // 4 :].T
    out_ref[...] = jnp.concatenate([lo, hi], axis=1)


def _relayout_tc(table_t):
    return pl.pallas_call(
        _relayout_body,
        grid=(_RGRID,),
        in_specs=[pl.BlockSpec((ID_DIM, _RL), lambda i: (0, i))],
        out_specs=pl.BlockSpec((_RL // 4, 2 * ID_DIM), lambda i: (i, 0)),
        out_shape=jax.ShapeDtypeStruct((_NPROWS, 2 * ID_DIM), jnp.int32),
    )(table_t)


def _gather_pairs_sc(pair_ids, table2):
    mesh = plsc.VectorSubcoreMesh(core_axis_name="c", subcore_axis_name="s")

    @functools.partial(
        pl.kernel,
        mesh=mesh,
        out_type=jax.ShapeDtypeStruct((BATCH, 2 * ID_DIM), jnp.int32),
        scratch_types=[
            pltpu.VMEM((_B_PER_W,), jnp.int32),
            pltpu.VMEM((_B_PER_W, 2 * ID_DIM), jnp.int32),
            pltpu.SemaphoreType.DMA,
        ],
    )
    def gather_kernel(idx_hbm, table_hbm, out_hbm, idx_v, rows_v, sem):
        wid = lax.axis_index("s") * _NC + lax.axis_index("c")
        base = wid * _B_PER_W
        pltpu.sync_copy(idx_hbm.at[pl.ds(base, _B_PER_W)], idx_v)
        copies = []
        for j in range(_B_PER_W // _IDX_CHUNK):
            sl = pl.ds(j * _IDX_CHUNK, _IDX_CHUNK)
            copies.append(
                pltpu.async_copy(table_hbm.at[idx_v.at[sl]], rows_v.at[sl], sem)
            )
        for c in copies:
            c.wait()
        pltpu.sync_copy(rows_v, out_hbm.at[pl.ds(base, _B_PER_W)])

    return gather_kernel(pair_ids, table2)


def _mlp_body(pair_ref, half_ref, par_ref, feat_ref, w1a_ref, w1b_ref, b1_ref,
              w2_ref, b2_ref, out_ref):
    x = pair_ref[...]
    h = half_ref[...]
    p = par_ref[...]
    lo_f = lax.bitcast_convert_type(lax.shift_left(x, 16), jnp.float32)
    hi_f = lax.bitcast_convert_type(
        jnp.bitwise_and(x, jnp.int32(-65536)), jnp.float32
    )
    sel = lo_f * (1.0 - p) + hi_f * p
    emb = sel[:, :ID_DIM] * (1.0 - h) + sel[:, ID_DIM:] * h
    z = jnp.dot(emb, w1a_ref[...], preferred_element_type=jnp.float32)
    z += jnp.dot(feat_ref[...], w1b_ref[...], preferred_element_type=jnp.float32)
    z = jnp.maximum(z + b1_ref[...], 0.0)
    out = jnp.dot(z, w2_ref[...], preferred_element_type=jnp.float32) + b2_ref[...]
    norm = jnp.sqrt(jnp.sum(out * out, axis=1, keepdims=True))
    out_ref[...] = out / jnp.maximum(norm, 1e-12)


_BB = 2048


def _mlp_tc(pairs, half, par, feats, w1a, w1b, b1, w2, b2):
    grid = (BATCH // _BB,)
    return pl.pallas_call(
        _mlp_body,
        grid=grid,
        in_specs=[
            pl.BlockSpec((_BB, 2 * ID_DIM), lambda i: (i, 0)),
            pl.BlockSpec((_BB, 1), lambda i: (i, 0)),
            pl.BlockSpec((_BB, 1), lambda i: (i, 0)),
            pl.BlockSpec((_BB, NUM_FEATS), lambda i: (i, 0)),
            pl.BlockSpec((ID_DIM, HIDDEN), lambda i: (0, 0)),
            pl.BlockSpec((NUM_FEATS, HIDDEN), lambda i: (0, 0)),
            pl.BlockSpec((1, HIDDEN), lambda i: (0, 0)),
            pl.BlockSpec((HIDDEN, EMBED_DIM), lambda i: (0, 0)),
            pl.BlockSpec((1, EMBED_DIM), lambda i: (0, 0)),
        ],
        out_specs=pl.BlockSpec((_BB, EMBED_DIM), lambda i: (i, 0)),
        out_shape=jax.ShapeDtypeStruct((BATCH, EMBED_DIM), jnp.float32),
    )(pairs, half, par, feats, w1a, w1b, b1, w2, b2)


def kernel(customer_ids, numerical_features, emb_table, W1, b1, W2, b2):
    ids = customer_ids.astype(jnp.int32)
    table_t = emb_table.T
    table2 = _relayout_tc(table_t)
    quad_ids = (ids // _RL) * (_RL // 4) + ids % (_RL // 4)
    half = ((ids % (_RL // 2)) >= _RL // 4).astype(jnp.float32).reshape(BATCH, 1)
    par = ((ids % _RL) >= _RL // 2).astype(jnp.float32).reshape(BATCH, 1)
    pairs = _gather_pairs_sc(quad_ids, table2)
    w1a = W1[:ID_DIM]
    w1b = W1[ID_DIM:]
    return _mlp_tc(
        pairs,
        half,
        par,
        numerical_features,
        w1a,
        w1b,
        b1.reshape(1, HIDDEN),
        W2,
        b2.reshape(1, EMBED_DIM),
    )


# speedup vs baseline: 3.5573x; 1.0466x over previous
import functools

import jax
import jax.numpy as jnp
from jax import lax
from jax.experimental import pallas as pl
from jax.experimental.pallas import tpu as pltpu
from jax.experimental.pallas import tpu_sc as plsc

BATCH = 16384
EMBED_DIM = 128
ID_DIM = 64
NUM_FEATS = 20
HIDDEN = 128
NUM_CUST = 1000000

_NC = 2
_NS = 16
_NW = _NC * _NS
_B_PER_W = BATCH // _NW
_IDX_CHUNK = 128

_RL = 32768
_RGRID = pl.cdiv(NUM_CUST, _RL)
_NPROWS = _RGRID * (_RL // 4)


def _relayout_body(t2_ref, out_ref):
    x = t2_ref[...].astype(jnp.bfloat16)
    bits = lax.bitcast_convert_type(x, jnp.uint16).astype(jnp.uint32)
    w = lax.bitcast_convert_type(
        bits[:, : _RL // 2] | (bits[:, _RL // 2 :] << 16), jnp.int32
    )
    lo = w[:, : _RL // 4].T
    hi = w[:, _RL // 4 :].T
    out_ref[...] = jnp.concatenate([lo, hi], axis=1)


def _relayout_tc(table_t):
    return pl.pallas_call(
        _relayout_body,
        grid=(_RGRID,),
        in_specs=[pl.BlockSpec((ID_DIM, _RL), lambda i: (0, i))],
        out_specs=pl.BlockSpec((_RL // 4, 2 * ID_DIM), lambda i: (i, 0)),
        out_shape=jax.ShapeDtypeStruct((_NPROWS, 2 * ID_DIM), jnp.int32),
    )(table_t)


def _gather_pairs_sc(pair_ids, table2):
    mesh = plsc.VectorSubcoreMesh(core_axis_name="c", subcore_axis_name="s")

    @functools.partial(
        pl.kernel,
        mesh=mesh,
        out_type=jax.ShapeDtypeStruct((BATCH, 2 * ID_DIM), jnp.int32),
        scratch_types=[
            pltpu.VMEM((_B_PER_W,), jnp.int32),
            pltpu.VMEM((_B_PER_W, 2 * ID_DIM), jnp.int32),
            pltpu.SemaphoreType.DMA,
        ],
    )
    def gather_kernel(idx_hbm, table_hbm, out_hbm, idx_v, rows_v, sem):
        wid = lax.axis_index("s") * _NC + lax.axis_index("c")
        base = wid * _B_PER_W
        pltpu.sync_copy(idx_hbm.at[pl.ds(base, _B_PER_W)], idx_v)
        copies = []
        for j in range(_B_PER_W // _IDX_CHUNK):
            sl = pl.ds(j * _IDX_CHUNK, _IDX_CHUNK)
            copies.append(
                pltpu.async_copy(table_hbm.at[idx_v.at[sl]], rows_v.at[sl], sem)
            )
        for c in copies:
            c.wait()
        pltpu.sync_copy(rows_v, out_hbm.at[pl.ds(base, _B_PER_W)])

    return gather_kernel(pair_ids, table2)


def _mlp_body(pair_ref, half_ref, par_ref, feat_ref, w1a_ref, w1b_ref, b1_ref,
              w2_ref, b2_ref, out_ref):
    x = pair_ref[...]
    h = half_ref[...]
    p = par_ref[...]
    lo_f = lax.bitcast_convert_type(lax.shift_left(x, 16), jnp.float32)
    hi_f = lax.bitcast_convert_type(
        jnp.bitwise_and(x, jnp.int32(-65536)), jnp.float32
    )
    sel = jnp.where(p > 0.0, hi_f, lo_f)
    emb = jnp.where(h > 0.0, sel[:, ID_DIM:], sel[:, :ID_DIM])
    z = jnp.dot(emb, w1a_ref[...], preferred_element_type=jnp.float32)
    z += lax.dot_general(
        feat_ref[...], w1b_ref[...], (((0,), (0,)), ((), ())),
        preferred_element_type=jnp.float32,
    )
    z = jnp.maximum(z + b1_ref[...], 0.0)
    out = jnp.dot(z, w2_ref[...], preferred_element_type=jnp.float32) + b2_ref[...]
    norm = jnp.sqrt(jnp.sum(out * out, axis=1, keepdims=True))
    out_ref[...] = out / jnp.maximum(norm, 1e-12)


_BB = 4096


def _mlp_tc(pairs, half, par, feats, w1a, w1b, b1, w2, b2):
    grid = (BATCH // _BB,)
    return pl.pallas_call(
        _mlp_body,
        grid=grid,
        in_specs=[
            pl.BlockSpec((_BB, 2 * ID_DIM), lambda i: (i, 0)),
            pl.BlockSpec((_BB, 1), lambda i: (i, 0)),
            pl.BlockSpec((_BB, 1), lambda i: (i, 0)),
            pl.BlockSpec((NUM_FEATS, _BB), lambda i: (0, i)),
            pl.BlockSpec((ID_DIM, HIDDEN), lambda i: (0, 0)),
            pl.BlockSpec((NUM_FEATS, HIDDEN), lambda i: (0, 0)),
            pl.BlockSpec((1, HIDDEN), lambda i: (0, 0)),
            pl.BlockSpec((HIDDEN, EMBED_DIM), lambda i: (0, 0)),
            pl.BlockSpec((1, EMBED_DIM), lambda i: (0, 0)),
        ],
        out_specs=pl.BlockSpec((_BB, EMBED_DIM), lambda i: (i, 0)),
        out_shape=jax.ShapeDtypeStruct((BATCH, EMBED_DIM), jnp.float32),
    )(pairs, half, par, feats, w1a, w1b, b1, w2, b2)


def kernel(customer_ids, numerical_features, emb_table, W1, b1, W2, b2):
    ids = customer_ids.astype(jnp.int32)
    table_t = emb_table.T
    table2 = _relayout_tc(table_t)
    quad_ids = (ids // _RL) * (_RL // 4) + ids % (_RL // 4)
    half = ((ids % (_RL // 2)) >= _RL // 4).astype(jnp.float32).reshape(BATCH, 1)
    par = ((ids % _RL) >= _RL // 2).astype(jnp.float32).reshape(BATCH, 1)
    pairs = _gather_pairs_sc(quad_ids, table2)
    w1a = W1[:ID_DIM]
    w1b = W1[ID_DIM:]
    return _mlp_tc(
        pairs,
        half,
        par,
        numerical_features.T,
        w1a,
        w1b,
        b1.reshape(1, HIDDEN),
        W2,
        b2.reshape(1, EMBED_DIM),
    )
